# baseline (speedup 1.0000x reference)
import functools
import math

import jax
import jax.numpy as jnp
from jax import lax
from jax.experimental import pallas as pl
from jax.experimental.pallas import tpu as pltpu
from jax.experimental.pallas import tpu_sc as plsc

N = 10000
E = 320000
H = 128

NCORES = 2
NSUB = 16
NTILES = NCORES * NSUB
EPT = E // NTILES
CHUNK = 128
GC = 40
NCH = 80
NG = NCH // GC
PADN = NCH * CHUNK - EPT
NP = 10112
RPT = NP // NSUB
BLK = 1000
GRID = N // BLK


def _mesh():
    return plsc.VectorSubcoreMesh(
        core_axis_name="c", subcore_axis_name="s",
        num_cores=NCORES, num_subcores=NSUB)


def _pad_idx(a, padval):
    a2 = a.reshape(NTILES, EPT)
    a2 = jnp.pad(a2, ((0, 0), (0, PADN)), constant_values=padval)
    return a2.reshape(NTILES, NCH, CHUNK)


def _agg(y, gidx, sidx, zrows):
    @functools.partial(
        pl.kernel,
        out_type=jax.ShapeDtypeStruct((NCORES, NP, H), jnp.float32),
        mesh=_mesh(),
        scratch_types=[
            pltpu.VMEM_SHARED((NP, H), jnp.float32),
            pltpu.VMEM((GC, CHUNK), jnp.int32),
            pltpu.VMEM((GC, CHUNK), jnp.int32),
            pltpu.VMEM((CHUNK, H), jnp.float32),
            pltpu.VMEM((CHUNK, H), jnp.float32),
            pltpu.SemaphoreType.DMA,
            pltpu.SemaphoreType.DMA,
            pltpu.SemaphoreType.DMA,
            pltpu.SemaphoreType.DMA,
        ],
    )
    def k(y_hbm, gi_hbm, si_hbm, z0_hbm, out_hbm,
          z_sh, gi_v, si_v, buf0, buf1, sg0, sg1, ss0, ss1):
        cid = lax.axis_index("c")
        sid = lax.axis_index("s")
        w = cid * NSUB + sid
        pltpu.sync_copy(z0_hbm, z_sh.at[pl.ds(sid * RPT, RPT)])
        plsc.subcore_barrier()

        def group(g, carry):
            pltpu.sync_copy(gi_hbm.at[w, pl.ds(g * GC, GC)], gi_v)
            pltpu.sync_copy(si_hbm.at[w, pl.ds(g * GC, GC)], si_v)
            pltpu.async_copy(y_hbm.at[gi_v.at[0]], buf0, sg0)
            pltpu.async_copy(y_hbm.at[gi_v.at[1]], buf1, sg1)

            @plsc.parallel_loop(0, GC // 2)
            def _chunk(i):
                jj = 2 * i
                pltpu.make_async_copy(y_hbm.at[gi_v.at[jj]], buf0, sg0).wait()
                pltpu.async_copy(buf0, z_sh.at[si_v.at[jj]], ss0, add=True)
                pltpu.make_async_copy(
                    y_hbm.at[gi_v.at[jj + 1]], buf1, sg1).wait()
                pltpu.async_copy(buf1, z_sh.at[si_v.at[jj + 1]], ss1, add=True)
                pltpu.make_async_copy(buf0, z_sh.at[si_v.at[jj]], ss0).wait()

                @pl.when(jj + 2 < GC)
                def _():
                    pltpu.async_copy(y_hbm.at[gi_v.at[jj + 2]], buf0, sg0)

                pltpu.make_async_copy(buf1, z_sh.at[si_v.at[jj + 1]], ss1).wait()

                @pl.when(jj + 3 < GC)
                def _():
                    pltpu.async_copy(y_hbm.at[gi_v.at[jj + 3]], buf1, sg1)

            return carry

        lax.fori_loop(0, NG, group, 0)
        plsc.subcore_barrier()
        pltpu.sync_copy(z_sh.at[pl.ds(sid * RPT, RPT)],
                        out_hbm.at[cid, pl.ds(sid * RPT, RPT)])

    return k(y, gidx, sidx, zrows)


def _hist_multi(streams, ind, zrows):
    nstr = len(streams)

    @functools.partial(
        pl.kernel,
        out_type=jax.ShapeDtypeStruct((NCORES, NP, H), jnp.float32),
        mesh=_mesh(),
        scratch_types=[
            pltpu.VMEM_SHARED((NP, H), jnp.float32),
            pltpu.VMEM((CHUNK, H), jnp.float32),
            pltpu.VMEM((GC, CHUNK), jnp.int32),
            pltpu.SemaphoreType.DMA,
            pltpu.SemaphoreType.DMA,
        ],
    )
    def k(*refs):
        s_hbms = refs[:nstr]
        ind_hbm, z_hbm, out_hbm, t_sh, buf, si_v, sem0, sem1 = refs[nstr:]
        cid = lax.axis_index("c")
        sid = lax.axis_index("s")
        w = cid * NSUB + sid
        pltpu.sync_copy(z_hbm, t_sh.at[pl.ds(sid * RPT, RPT)])
        plsc.subcore_barrier()

        for t, s_hbm in enumerate(s_hbms):
            pltpu.sync_copy(ind_hbm.at[t], buf)

            def group(g, carry):
                pltpu.sync_copy(s_hbm.at[w, pl.ds(g * GC, GC)], si_v)

                @plsc.parallel_loop(0, GC // 2)
                def _chunk(i):
                    jj = 2 * i

                    @pl.when(jj >= 2)
                    def _():
                        pltpu.make_async_copy(
                            buf, t_sh.at[si_v.at[jj]], sem0).wait()

                    pltpu.async_copy(buf, t_sh.at[si_v.at[jj]], sem0, add=True)

                    @pl.when(jj + 1 >= 2)
                    def _():
                        pltpu.make_async_copy(
                            buf, t_sh.at[si_v.at[jj + 1]], sem1).wait()

                    pltpu.async_copy(buf, t_sh.at[si_v.at[jj + 1]], sem1,
                                     add=True)
                pltpu.make_async_copy(buf, t_sh.at[si_v.at[0]], sem0).wait()
                pltpu.make_async_copy(buf, t_sh.at[si_v.at[1]], sem1).wait()
                return carry

            lax.fori_loop(0, NG, group, 0)

        plsc.subcore_barrier()
        pltpu.sync_copy(t_sh.at[pl.ds(sid * RPT, RPT)],
                        out_hbm.at[cid, pl.ds(sid * RPT, RPT)])

    return k(*streams, ind, zrows)


def _ln(x, g, b):
    m = jnp.mean(x, axis=-1, keepdims=True)
    v = jnp.mean((x - m) * (x - m), axis=-1, keepdims=True)
    return (x - m) * lax.rsqrt(v + 1e-5) * g + b


def _zspec():
    return pl.BlockSpec((NCORES, BLK, H), lambda i: (0, i, 0))


def _rspec(w=H):
    return pl.BlockSpec((BLK, w), lambda i: (i, 0))


def _dspec():
    return pl.BlockSpec((BLK, 1), lambda i: (i, 0))


def _wspec(r, c):
    return pl.BlockSpec((r, c), lambda i: (0, 0))


def _prep(hists):
    def body(h_ref, o_ref):
        hb = h_ref[...]
        s = hb[0, :N, :] + hb[1, :N, :]
        deg0 = s[:, 0]
        deg1 = s[:, 32]
        deg2 = s[:, 64]
        deg3 = s[:, 96]
        d0 = lax.rsqrt(deg0 + 1.0)
        d1 = lax.rsqrt(deg1 + 1.0)
        dD = jnp.where(deg2 > 0, 1.0 / jnp.where(deg2 > 0, deg2, 1.0), 0.0)
        dB = jnp.where(deg3 > 0, 1.0 / jnp.where(deg3 > 0, deg3, 1.0), 0.0)
        o_ref[...] = jnp.stack([d0, d1, dD, dB])

    return pl.pallas_call(
        body,
        out_shape=jax.ShapeDtypeStruct((4, N), jnp.float32),
    )(hists)


def _scale(x, d):
    def body(x_ref, d_ref, o_ref):
        o_ref[...] = x_ref[...] * d_ref[...]

    return pl.pallas_call(
        body, grid=(GRID,),
        in_specs=[_rspec(), _dspec()],
        out_specs=_rspec(),
        out_shape=jax.ShapeDtypeStruct((N, H), jnp.float32),
    )(x, d)


def _comb(z, d):
    def body(z_ref, d_ref, o_ref):
        zb = z_ref[...]
        o_ref[...] = (zb[0] + zb[1]) * d_ref[...]

    return pl.pallas_call(
        body, grid=(GRID,),
        in_specs=[_zspec(), _dspec()],
        out_specs=_rspec(),
        out_shape=jax.ShapeDtypeStruct((N, H), jnp.float32),
    )(z, d)


def _agg_mm(z, y, d, W, b, g, be):
    has_y = y is not None

    def body(*refs):
        if has_y:
            z_ref, y_ref, d_ref, w_ref, b_ref, g_ref, be_ref, o_ref = refs
        else:
            z_ref, d_ref, w_ref, b_ref, g_ref, be_ref, o_ref = refs
        zb = z_ref[...]
        t = zb[0] + zb[1]
        if has_y:
            t = t + refs[1][...]
        t = t * d_ref[...]
        h = jnp.dot(t, w_ref[...], preferred_element_type=jnp.float32)
        h = jax.nn.relu(h + b_ref[...])
        o_ref[...] = _ln(h, g_ref[...], be_ref[...])

    in_specs = [_zspec()]
    args = [z]
    if has_y:
        in_specs.append(_rspec())
        args.append(y)
    in_specs += [_dspec(), _wspec(H, 2 * H), _wspec(1, 2 * H),
                 _wspec(1, 2 * H), _wspec(1, 2 * H)]
    args += [d, W, b.reshape(1, -1), g.reshape(1, -1), be.reshape(1, -1)]
    return pl.pallas_call(
        body, grid=(GRID,),
        in_specs=in_specs,
        out_specs=_rspec(2 * H),
        out_shape=jax.ShapeDtypeStruct((N, 2 * H), jnp.float32),
    )(*args)


def _mm_scale(x, W, d):
    has_d = d is not None

    def body(*refs):
        if has_d:
            x_ref, w_ref, d_ref, o_ref = refs
        else:
            x_ref, w_ref, o_ref = refs
        h = jnp.dot(x_ref[...], w_ref[...], preferred_element_type=jnp.float32)
        if has_d:
            h = h * refs[2][...]
        o_ref[...] = h

    in_specs = [_rspec(2 * H), _wspec(2 * H, H)]
    args = [x, W]
    if has_d:
        in_specs.append(_dspec())
        args.append(d)
    return pl.pallas_call(
        body, grid=(GRID,),
        in_specs=in_specs,
        out_specs=_rspec(),
        out_shape=jax.ShapeDtypeStruct((N, H), jnp.float32),
    )(*args)


def _agg_post(z, y, d, b, g, be):
    has_y = y is not None

    def body(*refs):
        if has_y:
            z_ref, y_ref, d_ref, b_ref, g_ref, be_ref, o_ref = refs
        else:
            z_ref, d_ref, b_ref, g_ref, be_ref, o_ref = refs
        zb = z_ref[...]
        t = zb[0] + zb[1]
        if has_y:
            t = t + refs[1][...]
        t = jax.nn.relu(t * d_ref[...] + b_ref[...])
        o_ref[...] = _ln(t, g_ref[...], be_ref[...])

    in_specs = [_zspec()]
    args = [z]
    if has_y:
        in_specs.append(_rspec())
        args.append(y)
    in_specs += [_dspec(), _wspec(1, H), _wspec(1, H), _wspec(1, H)]
    args += [d, b.reshape(1, -1), g.reshape(1, -1), be.reshape(1, -1)]
    return pl.pallas_call(
        body, grid=(GRID,),
        in_specs=in_specs,
        out_specs=_rspec(),
        out_shape=jax.ShapeDtypeStruct((N, H), jnp.float32),
    )(*args)


def _attn(q, x1, x2, x3, g, b):
    inv = 1.0 / math.sqrt(H)

    def body(q_ref, x1_ref, x2_ref, x3_ref, g_ref, b_ref, o_ref):
        qb = q_ref[...]
        k1, k2, k3 = x1_ref[...], x2_ref[...], x3_ref[...]
        s1 = jnp.sum(qb * k1, axis=-1, keepdims=True) * inv
        s2 = jnp.sum(qb * k2, axis=-1, keepdims=True) * inv
        s3 = jnp.sum(qb * k3, axis=-1, keepdims=True) * inv
        m = jnp.maximum(jnp.maximum(s1, s2), s3)
        e1 = jnp.exp(s1 - m)
        e2 = jnp.exp(s2 - m)
        e3 = jnp.exp(s3 - m)
        den = e1 + e2 + e3
        out = (e1 * k1 + e2 * k2 + e3 * k3) / den
        o_ref[...] = _ln(out, g_ref[...], b_ref[...])

    return pl.pallas_call(
        body, grid=(GRID,),
        in_specs=[_rspec(), _rspec(), _rspec(), _rspec(),
                  _wspec(1, H), _wspec(1, H)],
        out_specs=_rspec(),
        out_shape=jax.ShapeDtypeStruct((N, H), jnp.float32),
    )(q, x1, x2, x3, g.reshape(1, -1), b.reshape(1, -1))


def _indicator(n):
    lane = jnp.arange(H, dtype=jnp.int32)
    ind = (jnp.arange(n, dtype=jnp.int32)[:, None] == (lane // 32)[None, :])
    return jnp.broadcast_to(ind.astype(jnp.float32)[:, None, :], (n, CHUNK, H))


def _prep2(hists):
    def body(h_ref, o_ref):
        hb = h_ref[...]
        c = hb[0, :N, :] + hb[1, :N, :]
        c0 = c[:, 0]
        c1 = c[:, 32]
        r0 = lax.rsqrt(c0 + 1.0)
        r1 = lax.rsqrt(c1 + 1.0)
        m0 = jnp.where(c0 > 0, 1.0 / jnp.where(c0 > 0, c0, 1.0), 0.0)
        m1 = jnp.where(c1 > 0, 1.0 / jnp.where(c1 > 0, c1, 1.0), 0.0)
        o_ref[...] = jnp.stack([r0, r1, m0, m1])

    return pl.pallas_call(
        body, out_shape=jax.ShapeDtypeStruct((4, N), jnp.float32))(hists)


def _gcn_branch(emb, d, gi, si, zrows,
                W1, b1, g1, be1, W2, b2, g2, be2):
    y = _scale(emb, d)
    z = _agg(y, gi, si, zrows)
    h1 = _agg_mm(z, y, d, W1, b1, g1, be1)
    y2 = _mm_scale(h1, W2, d)
    z2 = _agg(y2, gi, si, zrows)
    return _agg_post(z2, y2, d, b2, g2, be2)


def _hconv_branch(emb, dD, dB, gia, sia, gib, sib, zrows,
                  W1, b1, g1, be1, W2, b2, g2, be2):
    za = _agg(emb, gia, sia, zrows)
    oe = _comb(za, dB)
    zb = _agg(oe, gib, sib, zrows)
    h3 = _agg_mm(zb, None, dD, W1, b1, g1, be1)
    hw = _mm_scale(h3, W2, None)
    zc = _agg(hw, gia, sia, zrows)
    oe2 = _comb(zc, dB)
    zd = _agg(oe2, gib, sib, zrows)
    return _agg_post(zd, None, dD, b2, g2, be2)


def _pad_all(sg):
    return (_pad_idx(sg[0, 0], 0), _pad_idx(sg[0, 1], N),
            _pad_idx(sg[1, 0], 0), _pad_idx(sg[1, 1], N),
            _pad_idx(sg[2, 0], 0), _pad_idx(sg[2, 1], N),
            _pad_idx(sg[2, 1], 0), _pad_idx(sg[2, 0], N))


def _kernel_1dev(static_graphs, emb,
                 l1_W1, l1_b1, l1_W2, l1_b2, l1_g1, l1_be1, l1_g2, l1_be2,
                 l2_W1, l2_b1, l2_W2, l2_b2, l2_g1, l2_be1, l2_g2, l2_be2,
                 l3_W1, l3_b1, l3_W2, l3_b2, l3_g1, l3_be1, l3_g2, l3_be2,
                 norm_g, norm_b):
    gi0, si0, gi1, si1, gi2a, si2a, gi2b, si2b = _pad_all(static_graphs)
    zrows = jnp.zeros((RPT, H), jnp.float32)

    hists = _hist_multi([si0, si1, si2b, si2a], _indicator(4), zrows)
    dinv = _prep(hists)
    d0 = dinv[0].reshape(N, 1)
    d1 = dinv[1].reshape(N, 1)
    dD = dinv[2].reshape(N, 1)
    dB = dinv[3].reshape(N, 1)

    x1 = _gcn_branch(emb, d0, gi0, si0, zrows, l1_W1, l1_b1, l1_g1, l1_be1,
                     l1_W2, l1_b2, l1_g2, l1_be2)
    x2 = _gcn_branch(emb, d1, gi1, si1, zrows, l2_W1, l2_b1, l2_g1, l2_be1,
                     l2_W2, l2_b2, l2_g2, l2_be2)
    x3 = _hconv_branch(emb, dD, dB, gi2a, si2a, gi2b, si2b, zrows,
                       l3_W1, l3_b1, l3_g1, l3_be1, l3_W2, l3_b2, l3_g2, l3_be2)

    return _attn(emb, x1, x2, x3, norm_g, norm_b)


def _kernel_2dev(static_graphs, emb,
                 l1_W1, l1_b1, l1_W2, l1_b2, l1_g1, l1_be1, l1_g2, l1_be2,
                 l2_W1, l2_b1, l2_W2, l2_b2, l2_g1, l2_be1, l2_g2, l2_be2,
                 l3_W1, l3_b1, l3_W2, l3_b2, l3_g1, l3_be1, l3_g2, l3_be2,
                 norm_g, norm_b):
    idx = _pad_all(static_graphs)

    def body(emb, gi0, si0, gi1, si1, gi2a, si2a, gi2b, si2b,
             a_W1, a_b1, a_W2, a_b2, a_g1, a_be1, a_g2, a_be2,
             b_W1, b_b1, b_W2, b_b2, b_g1, b_be1, b_g2, b_be2,
             c_W1, c_b1, c_W2, c_b2, c_g1, c_be1, c_g2, c_be2,
             ng, nb):
        did = lax.axis_index("d")
        zrows = jnp.zeros((RPT, H), jnp.float32)
        ind2 = _indicator(2)

        def gcn_pair(_):
            hists = _hist_multi([si0, si1], ind2, zrows)
            pr = _prep2(hists)
            d0 = pr[0].reshape(N, 1)
            d1 = pr[1].reshape(N, 1)
            x1 = _gcn_branch(emb, d0, gi0, si0, zrows,
                             a_W1, a_b1, a_g1, a_be1, a_W2, a_b2, a_g2, a_be2)
            x2 = _gcn_branch(emb, d1, gi1, si1, zrows,
                             b_W1, b_b1, b_g1, b_be1, b_W2, b_b2, b_g2, b_be2)
            return x1, x2, jnp.zeros((N, H), jnp.float32)

        def hconv_one(_):
            hists = _hist_multi([si2b, si2a], ind2, zrows)
            pr = _prep2(hists)
            dD = pr[2].reshape(N, 1)
            dB = pr[3].reshape(N, 1)
            x3 = _hconv_branch(emb, dD, dB, gi2a, si2a, gi2b, si2b, zrows,
                               c_W1, c_b1, c_g1, c_be1, c_W2, c_b2, c_g2, c_be2)
            zero = jnp.zeros((N, H), jnp.float32)
            return zero, zero, x3

        x1, x2, x3 = lax.cond(did == 0, gcn_pair, hconv_one, 0)
        x1 = lax.psum(x1, "d")
        x2 = lax.psum(x2, "d")
        x3 = lax.psum(x3, "d")
        return _attn(emb, x1, x2, x3, ng, nb)

    args = (emb, *idx,
            l1_W1, l1_b1, l1_W2, l1_b2, l1_g1, l1_be1, l1_g2, l1_be2,
            l2_W1, l2_b1, l2_W2, l2_b2, l2_g1, l2_be1, l2_g2, l2_be2,
            l3_W1, l3_b1, l3_W2, l3_b2, l3_g1, l3_be1, l3_g2, l3_be2,
            norm_g, norm_b)
    mesh = jax.make_mesh((2,), ("d",))
    P = jax.sharding.PartitionSpec
    f = jax.shard_map(body, mesh=mesh, in_specs=(P(),) * len(args),
                      out_specs=P(), check_vma=False)
    return f(*args)


def kernel(static_graphs, emb,
           l1_W1, l1_b1, l1_W2, l1_b2, l1_g1, l1_be1, l1_g2, l1_be2,
           l2_W1, l2_b1, l2_W2, l2_b2, l2_g1, l2_be1, l2_g2, l2_be2,
           l3_W1, l3_b1, l3_W2, l3_b2, l3_g1, l3_be1, l3_g2, l3_be2,
           norm_g, norm_b):
    args = (static_graphs, emb,
            l1_W1, l1_b1, l1_W2, l1_b2, l1_g1, l1_be1, l1_g2, l1_be2,
            l2_W1, l2_b1, l2_W2, l2_b2, l2_g1, l2_be1, l2_g2, l2_be2,
            l3_W1, l3_b1, l3_W2, l3_b2, l3_g1, l3_be1, l3_g2, l3_be2,
            norm_g, norm_b)
    if len(jax.devices()) >= 2:
        return _kernel_2dev(*args)
    return _kernel_1dev(*args)


# speedup vs baseline: 1.0831x; 1.0831x over previous
import functools
import math

import jax
import jax.numpy as jnp
from jax import lax
from jax.experimental import pallas as pl
from jax.experimental.pallas import tpu as pltpu
from jax.experimental.pallas import tpu_sc as plsc

N = 10000
E = 320000
H = 128

NCORES = 2
NSUB = 16
NTILES = NCORES * NSUB
EPT = E // NTILES
CHUNK = 128
GC = 40
NCH = 80
NG = NCH // GC
PADN = NCH * CHUNK - EPT
NP = 10112
RPT = NP // NSUB
BLK = 1000
GRID = N // BLK


def _mesh():
    return plsc.VectorSubcoreMesh(
        core_axis_name="c", subcore_axis_name="s",
        num_cores=NCORES, num_subcores=NSUB)


def _pad_idx(a, padval):
    a2 = a.reshape(NTILES, EPT)
    a2 = jnp.pad(a2, ((0, 0), (0, PADN)), constant_values=padval)
    return a2.reshape(NTILES, NCH, CHUNK)


def _agg(y, gidx, sidx, zrows):
    @functools.partial(
        pl.kernel,
        out_type=jax.ShapeDtypeStruct((NCORES, NP, H), jnp.float32),
        mesh=_mesh(),
        scratch_types=[
            pltpu.VMEM_SHARED((NP, H), jnp.float32),
            pltpu.VMEM((GC, CHUNK), jnp.int32),
            pltpu.VMEM((GC, CHUNK), jnp.int32),
            pltpu.VMEM((CHUNK, H), jnp.float32),
            pltpu.VMEM((CHUNK, H), jnp.float32),
            pltpu.SemaphoreType.DMA,
            pltpu.SemaphoreType.DMA,
        ],
    )
    def k(y_hbm, gi_hbm, si_hbm, z0_hbm, out_hbm,
          z_sh, gi_v, si_v, buf0, buf1, sg0, sg1):
        cid = lax.axis_index("c")
        sid = lax.axis_index("s")
        w = cid * NSUB + sid
        pltpu.sync_copy(z0_hbm, z_sh.at[pl.ds(sid * RPT, RPT)])
        plsc.subcore_barrier()

        def group(g, carry):
            pltpu.sync_copy(gi_hbm.at[w, pl.ds(g * GC, GC)], gi_v)
            pltpu.sync_copy(si_hbm.at[w, pl.ds(g * GC, GC)], si_v)
            pltpu.async_copy(y_hbm.at[gi_v.at[0]], buf0, sg0)
            pltpu.async_copy(y_hbm.at[gi_v.at[1]], buf1, sg1)

            def one(jj, buf, sem):
                pltpu.make_async_copy(y_hbm.at[gi_v.at[jj]], buf, sem).wait()
                pltpu.sync_copy(buf, z_sh.at[si_v.at[jj]], add=True)

                @pl.when(jj + 2 < GC)
                def _():
                    pltpu.async_copy(y_hbm.at[gi_v.at[jj + 2]], buf, sem)

            @plsc.parallel_loop(0, GC // 2)
            def _chunk(i):
                jj = 2 * i
                one(jj, buf0, sg0)
                one(jj + 1, buf1, sg1)

            return carry

        lax.fori_loop(0, NG, group, 0)
        plsc.subcore_barrier()
        pltpu.sync_copy(z_sh.at[pl.ds(sid * RPT, RPT)],
                        out_hbm.at[cid, pl.ds(sid * RPT, RPT)])

    return k(y, gidx, sidx, zrows)


def _hist_multi(streams, ind, zrows):
    nstr = len(streams)

    @functools.partial(
        pl.kernel,
        out_type=jax.ShapeDtypeStruct((NCORES, NP, H), jnp.float32),
        mesh=_mesh(),
        scratch_types=[
            pltpu.VMEM_SHARED((NP, H), jnp.float32),
            pltpu.VMEM((CHUNK, H), jnp.float32),
            pltpu.VMEM((GC, CHUNK), jnp.int32),
            pltpu.SemaphoreType.DMA,
            pltpu.SemaphoreType.DMA,
        ],
    )
    def k(*refs):
        s_hbms = refs[:nstr]
        ind_hbm, z_hbm, out_hbm, t_sh, buf, si_v, sem0, sem1 = refs[nstr:]
        cid = lax.axis_index("c")
        sid = lax.axis_index("s")
        w = cid * NSUB + sid
        pltpu.sync_copy(z_hbm, t_sh.at[pl.ds(sid * RPT, RPT)])
        plsc.subcore_barrier()

        for t, s_hbm in enumerate(s_hbms):
            pltpu.sync_copy(ind_hbm.at[t], buf)

            def group(g, carry):
                pltpu.sync_copy(s_hbm.at[w, pl.ds(g * GC, GC)], si_v)

                @plsc.parallel_loop(0, GC // 2)
                def _chunk(i):
                    jj = 2 * i

                    @pl.when(jj >= 2)
                    def _():
                        pltpu.make_async_copy(
                            buf, t_sh.at[si_v.at[jj]], sem0).wait()

                    pltpu.async_copy(buf, t_sh.at[si_v.at[jj]], sem0, add=True)

                    @pl.when(jj + 1 >= 2)
                    def _():
                        pltpu.make_async_copy(
                            buf, t_sh.at[si_v.at[jj + 1]], sem1).wait()

                    pltpu.async_copy(buf, t_sh.at[si_v.at[jj + 1]], sem1,
                                     add=True)
                pltpu.make_async_copy(buf, t_sh.at[si_v.at[0]], sem0).wait()
                pltpu.make_async_copy(buf, t_sh.at[si_v.at[1]], sem1).wait()
                return carry

            lax.fori_loop(0, NG, group, 0)

        plsc.subcore_barrier()
        pltpu.sync_copy(t_sh.at[pl.ds(sid * RPT, RPT)],
                        out_hbm.at[cid, pl.ds(sid * RPT, RPT)])

    return k(*streams, ind, zrows)


def _ln(x, g, b):
    m = jnp.mean(x, axis=-1, keepdims=True)
    v = jnp.mean((x - m) * (x - m), axis=-1, keepdims=True)
    return (x - m) * lax.rsqrt(v + 1e-5) * g + b


def _zspec():
    return pl.BlockSpec((NCORES, BLK, H), lambda i: (0, i, 0))


def _rspec(w=H):
    return pl.BlockSpec((BLK, w), lambda i: (i, 0))


def _dspec():
    return pl.BlockSpec((BLK, 1), lambda i: (i, 0))


def _wspec(r, c):
    return pl.BlockSpec((r, c), lambda i: (0, 0))


def _prep(hists):
    def body(h_ref, o_ref):
        hb = h_ref[...]
        s = hb[0, :N, :] + hb[1, :N, :]
        deg0 = s[:, 0]
        deg1 = s[:, 32]
        deg2 = s[:, 64]
        deg3 = s[:, 96]
        d0 = lax.rsqrt(deg0 + 1.0)
        d1 = lax.rsqrt(deg1 + 1.0)
        dD = jnp.where(deg2 > 0, 1.0 / jnp.where(deg2 > 0, deg2, 1.0), 0.0)
        dB = jnp.where(deg3 > 0, 1.0 / jnp.where(deg3 > 0, deg3, 1.0), 0.0)
        o_ref[...] = jnp.stack([d0, d1, dD, dB])

    return pl.pallas_call(
        body,
        out_shape=jax.ShapeDtypeStruct((4, N), jnp.float32),
    )(hists)


def _scale(x, d):
    def body(x_ref, d_ref, o_ref):
        o_ref[...] = x_ref[...] * d_ref[...]

    return pl.pallas_call(
        body, grid=(GRID,),
        in_specs=[_rspec(), _dspec()],
        out_specs=_rspec(),
        out_shape=jax.ShapeDtypeStruct((N, H), jnp.float32),
    )(x, d)


def _comb(z, d):
    def body(z_ref, d_ref, o_ref):
        zb = z_ref[...]
        o_ref[...] = (zb[0] + zb[1]) * d_ref[...]

    return pl.pallas_call(
        body, grid=(GRID,),
        in_specs=[_zspec(), _dspec()],
        out_specs=_rspec(),
        out_shape=jax.ShapeDtypeStruct((N, H), jnp.float32),
    )(z, d)


def _agg_mm(z, y, d, W, b, g, be):
    has_y = y is not None

    def body(*refs):
        if has_y:
            z_ref, y_ref, d_ref, w_ref, b_ref, g_ref, be_ref, o_ref = refs
        else:
            z_ref, d_ref, w_ref, b_ref, g_ref, be_ref, o_ref = refs
        zb = z_ref[...]
        t = zb[0] + zb[1]
        if has_y:
            t = t + refs[1][...]
        t = t * d_ref[...]
        h = jnp.dot(t, w_ref[...], preferred_element_type=jnp.float32)
        h = jax.nn.relu(h + b_ref[...])
        o_ref[...] = _ln(h, g_ref[...], be_ref[...])

    in_specs = [_zspec()]
    args = [z]
    if has_y:
        in_specs.append(_rspec())
        args.append(y)
    in_specs += [_dspec(), _wspec(H, 2 * H), _wspec(1, 2 * H),
                 _wspec(1, 2 * H), _wspec(1, 2 * H)]
    args += [d, W, b.reshape(1, -1), g.reshape(1, -1), be.reshape(1, -1)]
    return pl.pallas_call(
        body, grid=(GRID,),
        in_specs=in_specs,
        out_specs=_rspec(2 * H),
        out_shape=jax.ShapeDtypeStruct((N, 2 * H), jnp.float32),
    )(*args)


def _mm_scale(x, W, d):
    has_d = d is not None

    def body(*refs):
        if has_d:
            x_ref, w_ref, d_ref, o_ref = refs
        else:
            x_ref, w_ref, o_ref = refs
        h = jnp.dot(x_ref[...], w_ref[...], preferred_element_type=jnp.float32)
        if has_d:
            h = h * refs[2][...]
        o_ref[...] = h

    in_specs = [_rspec(2 * H), _wspec(2 * H, H)]
    args = [x, W]
    if has_d:
        in_specs.append(_dspec())
        args.append(d)
    return pl.pallas_call(
        body, grid=(GRID,),
        in_specs=in_specs,
        out_specs=_rspec(),
        out_shape=jax.ShapeDtypeStruct((N, H), jnp.float32),
    )(*args)


def _agg_post(z, y, d, b, g, be):
    has_y = y is not None

    def body(*refs):
        if has_y:
            z_ref, y_ref, d_ref, b_ref, g_ref, be_ref, o_ref = refs
        else:
            z_ref, d_ref, b_ref, g_ref, be_ref, o_ref = refs
        zb = z_ref[...]
        t = zb[0] + zb[1]
        if has_y:
            t = t + refs[1][...]
        t = jax.nn.relu(t * d_ref[...] + b_ref[...])
        o_ref[...] = _ln(t, g_ref[...], be_ref[...])

    in_specs = [_zspec()]
    args = [z]
    if has_y:
        in_specs.append(_rspec())
        args.append(y)
    in_specs += [_dspec(), _wspec(1, H), _wspec(1, H), _wspec(1, H)]
    args += [d, b.reshape(1, -1), g.reshape(1, -1), be.reshape(1, -1)]
    return pl.pallas_call(
        body, grid=(GRID,),
        in_specs=in_specs,
        out_specs=_rspec(),
        out_shape=jax.ShapeDtypeStruct((N, H), jnp.float32),
    )(*args)


def _attn(q, x1, x2, x3, g, b):
    inv = 1.0 / math.sqrt(H)

    def body(q_ref, x1_ref, x2_ref, x3_ref, g_ref, b_ref, o_ref):
        qb = q_ref[...]
        k1, k2, k3 = x1_ref[...], x2_ref[...], x3_ref[...]
        s1 = jnp.sum(qb * k1, axis=-1, keepdims=True) * inv
        s2 = jnp.sum(qb * k2, axis=-1, keepdims=True) * inv
        s3 = jnp.sum(qb * k3, axis=-1, keepdims=True) * inv
        m = jnp.maximum(jnp.maximum(s1, s2), s3)
        e1 = jnp.exp(s1 - m)
        e2 = jnp.exp(s2 - m)
        e3 = jnp.exp(s3 - m)
        den = e1 + e2 + e3
        out = (e1 * k1 + e2 * k2 + e3 * k3) / den
        o_ref[...] = _ln(out, g_ref[...], b_ref[...])

    return pl.pallas_call(
        body, grid=(GRID,),
        in_specs=[_rspec(), _rspec(), _rspec(), _rspec(),
                  _wspec(1, H), _wspec(1, H)],
        out_specs=_rspec(),
        out_shape=jax.ShapeDtypeStruct((N, H), jnp.float32),
    )(q, x1, x2, x3, g.reshape(1, -1), b.reshape(1, -1))


def _indicator(n):
    lane = jnp.arange(H, dtype=jnp.int32)
    ind = (jnp.arange(n, dtype=jnp.int32)[:, None] == (lane // 32)[None, :])
    return jnp.broadcast_to(ind.astype(jnp.float32)[:, None, :], (n, CHUNK, H))


def _prep2(hists):
    def body(h_ref, o_ref):
        hb = h_ref[...]
        c = hb[0, :N, :] + hb[1, :N, :]
        c0 = c[:, 0]
        c1 = c[:, 32]
        r0 = lax.rsqrt(c0 + 1.0)
        r1 = lax.rsqrt(c1 + 1.0)
        m0 = jnp.where(c0 > 0, 1.0 / jnp.where(c0 > 0, c0, 1.0), 0.0)
        m1 = jnp.where(c1 > 0, 1.0 / jnp.where(c1 > 0, c1, 1.0), 0.0)
        o_ref[...] = jnp.stack([r0, r1, m0, m1])

    return pl.pallas_call(
        body, out_shape=jax.ShapeDtypeStruct((4, N), jnp.float32))(hists)


def _gcn_branch(emb, d, gi, si, zrows,
                W1, b1, g1, be1, W2, b2, g2, be2):
    y = _scale(emb, d)
    z = _agg(y, gi, si, zrows)
    h1 = _agg_mm(z, y, d, W1, b1, g1, be1)
    y2 = _mm_scale(h1, W2, d)
    z2 = _agg(y2, gi, si, zrows)
    return _agg_post(z2, y2, d, b2, g2, be2)


def _hconv_branch(emb, dD, dB, gia, sia, gib, sib, zrows,
                  W1, b1, g1, be1, W2, b2, g2, be2):
    za = _agg(emb, gia, sia, zrows)
    oe = _comb(za, dB)
    zb = _agg(oe, gib, sib, zrows)
    h3 = _agg_mm(zb, None, dD, W1, b1, g1, be1)
    hw = _mm_scale(h3, W2, None)
    zc = _agg(hw, gia, sia, zrows)
    oe2 = _comb(zc, dB)
    zd = _agg(oe2, gib, sib, zrows)
    return _agg_post(zd, None, dD, b2, g2, be2)


def _pad_all(sg):
    return (_pad_idx(sg[0, 0], 0), _pad_idx(sg[0, 1], N),
            _pad_idx(sg[1, 0], 0), _pad_idx(sg[1, 1], N),
            _pad_idx(sg[2, 0], 0), _pad_idx(sg[2, 1], N),
            _pad_idx(sg[2, 1], 0), _pad_idx(sg[2, 0], N))


def _kernel_1dev(static_graphs, emb,
                 l1_W1, l1_b1, l1_W2, l1_b2, l1_g1, l1_be1, l1_g2, l1_be2,
                 l2_W1, l2_b1, l2_W2, l2_b2, l2_g1, l2_be1, l2_g2, l2_be2,
                 l3_W1, l3_b1, l3_W2, l3_b2, l3_g1, l3_be1, l3_g2, l3_be2,
                 norm_g, norm_b):
    gi0, si0, gi1, si1, gi2a, si2a, gi2b, si2b = _pad_all(static_graphs)
    zrows = jnp.zeros((RPT, H), jnp.float32)

    hists = _hist_multi([si0, si1, si2b, si2a], _indicator(4), zrows)
    dinv = _prep(hists)
    d0 = dinv[0].reshape(N, 1)
    d1 = dinv[1].reshape(N, 1)
    dD = dinv[2].reshape(N, 1)
    dB = dinv[3].reshape(N, 1)

    x1 = _gcn_branch(emb, d0, gi0, si0, zrows, l1_W1, l1_b1, l1_g1, l1_be1,
                     l1_W2, l1_b2, l1_g2, l1_be2)
    x2 = _gcn_branch(emb, d1, gi1, si1, zrows, l2_W1, l2_b1, l2_g1, l2_be1,
                     l2_W2, l2_b2, l2_g2, l2_be2)
    x3 = _hconv_branch(emb, dD, dB, gi2a, si2a, gi2b, si2b, zrows,
                       l3_W1, l3_b1, l3_g1, l3_be1, l3_W2, l3_b2, l3_g2, l3_be2)

    return _attn(emb, x1, x2, x3, norm_g, norm_b)


def _kernel_2dev(static_graphs, emb,
                 l1_W1, l1_b1, l1_W2, l1_b2, l1_g1, l1_be1, l1_g2, l1_be2,
                 l2_W1, l2_b1, l2_W2, l2_b2, l2_g1, l2_be1, l2_g2, l2_be2,
                 l3_W1, l3_b1, l3_W2, l3_b2, l3_g1, l3_be1, l3_g2, l3_be2,
                 norm_g, norm_b):
    idx = _pad_all(static_graphs)

    def body(emb, gi0, si0, gi1, si1, gi2a, si2a, gi2b, si2b,
             a_W1, a_b1, a_W2, a_b2, a_g1, a_be1, a_g2, a_be2,
             b_W1, b_b1, b_W2, b_b2, b_g1, b_be1, b_g2, b_be2,
             c_W1, c_b1, c_W2, c_b2, c_g1, c_be1, c_g2, c_be2,
             ng, nb):
        did = lax.axis_index("d")
        zrows = jnp.zeros((RPT, H), jnp.float32)
        ind2 = _indicator(2)

        def gcn_pair(_):
            hists = _hist_multi([si0, si1], ind2, zrows)
            pr = _prep2(hists)
            d0 = pr[0].reshape(N, 1)
            d1 = pr[1].reshape(N, 1)
            x1 = _gcn_branch(emb, d0, gi0, si0, zrows,
                             a_W1, a_b1, a_g1, a_be1, a_W2, a_b2, a_g2, a_be2)
            x2 = _gcn_branch(emb, d1, gi1, si1, zrows,
                             b_W1, b_b1, b_g1, b_be1, b_W2, b_b2, b_g2, b_be2)
            return x1, x2, jnp.zeros((N, H), jnp.float32)

        def hconv_one(_):
            hists = _hist_multi([si2b, si2a], ind2, zrows)
            pr = _prep2(hists)
            dD = pr[2].reshape(N, 1)
            dB = pr[3].reshape(N, 1)
            x3 = _hconv_branch(emb, dD, dB, gi2a, si2a, gi2b, si2b, zrows,
                               c_W1, c_b1, c_g1, c_be1, c_W2, c_b2, c_g2, c_be2)
            zero = jnp.zeros((N, H), jnp.float32)
            return zero, zero, x3

        x1, x2, x3 = lax.cond(did == 0, gcn_pair, hconv_one, 0)
        x1 = lax.psum(x1, "d")
        x2 = lax.psum(x2, "d")
        x3 = lax.psum(x3, "d")
        return _attn(emb, x1, x2, x3, ng, nb)

    args = (emb, *idx,
            l1_W1, l1_b1, l1_W2, l1_b2, l1_g1, l1_be1, l1_g2, l1_be2,
            l2_W1, l2_b1, l2_W2, l2_b2, l2_g1, l2_be1, l2_g2, l2_be2,
            l3_W1, l3_b1, l3_W2, l3_b2, l3_g1, l3_be1, l3_g2, l3_be2,
            norm_g, norm_b)
    mesh = jax.make_mesh((2,), ("d",))
    P = jax.sharding.PartitionSpec
    f = jax.shard_map(body, mesh=mesh, in_specs=(P(),) * len(args),
                      out_specs=P(), check_vma=False)
    return f(*args)


def kernel(static_graphs, emb,
           l1_W1, l1_b1, l1_W2, l1_b2, l1_g1, l1_be1, l1_g2, l1_be2,
           l2_W1, l2_b1, l2_W2, l2_b2, l2_g1, l2_be1, l2_g2, l2_be2,
           l3_W1, l3_b1, l3_W2, l3_b2, l3_g1, l3_be1, l3_g2, l3_be2,
           norm_g, norm_b):
    args = (static_graphs, emb,
            l1_W1, l1_b1, l1_W2, l1_b2, l1_g1, l1_be1, l1_g2, l1_be2,
            l2_W1, l2_b1, l2_W2, l2_b2, l2_g1, l2_be1, l2_g2, l2_be2,
            l3_W1, l3_b1, l3_W2, l3_b2, l3_g1, l3_be1, l3_g2, l3_be2,
            norm_g, norm_b)
    if len(jax.devices()) >= 2:
        return _kernel_2dev(*args)
    return _kernel_1dev(*args)
